# baseline (speedup 1.0000x reference)
import functools

import jax
import jax.numpy as jnp
from jax import lax
from jax.experimental import pallas as pl
from jax.experimental.pallas import tpu as pltpu
from jax.experimental.pallas import tpu_sc as plsc

B = 4096
D0, D1, D2, DF = 32, 64, 64, 64
V0, V1, V2 = 1000, 100000, 1000000
NC, NS, L = 2, 16, 16
NW = NC * NS
BPW = B // NW
TAILBASE = (V2 // 128) * 128
NTAIL = V2 - TAILBASE
TCMAX = V2 // 128 - 1
G = 4


def _matvec(d, v, blk):
    grid = (v + blk - 1) // blk

    def body(w_ref, t_ref, o_ref):
        o_ref[...] = jnp.dot(
            w_ref[...], t_ref[...], preferred_element_type=jnp.float32
        )[0]

    return pl.pallas_call(
        body,
        grid=(grid,),
        in_specs=[
            pl.BlockSpec((1, d), lambda i: (0, 0)),
            pl.BlockSpec((d, blk), lambda i: (0, i)),
        ],
        out_specs=pl.BlockSpec((blk,), lambda i: (i,)),
        out_shape=jax.ShapeDtypeStruct((v,), jnp.float32),
    )


_MV0 = _matvec(D0, V0, 1024)
_MV1 = _matvec(D1, V1, 32768)
_MVF = _matvec(DF, B, 4096)
_MVT = _matvec(D2, NTAIL, NTAIL)


def _build_sc():
    mesh = plsc.VectorSubcoreMesh(core_axis_name="c", subcore_axis_name="s")

    @functools.partial(
        pl.kernel,
        mesh=mesh,
        out_type=jax.ShapeDtypeStruct((B,), jnp.float32),
        compiler_params=pltpu.CompilerParams(
            needs_layout_passes=False, use_tc_tiling_on_sc=True),
        scratch_types=[
            pltpu.VMEM((BPW + L,), jnp.int32),
            pltpu.VMEM((BPW,), jnp.int32),
            pltpu.VMEM((BPW,), jnp.int32),
            pltpu.VMEM((BPW,), jnp.int32),
            pltpu.VMEM((2 * G, D2, 128), jnp.float32),
            pltpu.VMEM((D2,), jnp.float32),
            pltpu.VMEM((BPW * L,), jnp.float32),
            pltpu.VMEM((BPW,), jnp.float32),
            pltpu.VMEM((BPW,), jnp.float32),
            pltpu.VMEM((BPW,), jnp.float32),
            pltpu.VMEM((BPW,), jnp.float32),
            pltpu.VMEM((L,), jnp.float32),
            pltpu.VMEM((BPW,), jnp.float32),
            pltpu.SemaphoreType.DMA,
            pltpu.SemaphoreType.DMA,
            pltpu.SemaphoreType.DMA,
        ],
    )
    def sc_kernel(id0_h, id1_h, id2_h, t2_h, w2_h, p0_h, p1_h, ptail_h, fd_h,
                  b_h, out_h, idv, idx0, idx1, idxt, stage, wv, accT, g0, g1,
                  gt, fdv, bv, outv, sem_a, sem_b, sem_g):
        wid = lax.axis_index("s") * NC + lax.axis_index("c")
        base = wid * BPW
        pltpu.sync_copy(id2_h.at[pl.ds(base, BPW)], idv.at[pl.ds(0, BPW)])
        pltpu.sync_copy(id0_h.at[pl.ds(base, BPW)], idx0)
        pltpu.sync_copy(id1_h.at[pl.ds(base, BPW)], idx1)
        for c in range(BPW // L):
            sl = pl.ds(c * L, L)
            idxt[sl] = jnp.clip(idv[sl] - TAILBASE, 0, NTAIL - 1)
        pltpu.async_copy(p0_h.at[idx0], g0, sem_g)
        pltpu.async_copy(p1_h.at[idx1], g1, sem_g)
        pltpu.async_copy(ptail_h.at[idxt], gt, sem_g)
        pltpu.sync_copy(w2_h, wv)
        wch = [wv[pl.ds(c * L, L)] for c in range(D2 // L)]
        iot = lax.iota(jnp.int32, L)
        row_base = iot * L

        def scol(s):
            tc = jnp.minimum(
                lax.shift_right_logical(s, 7), jnp.int32(TCMAX))
            return pl.multiple_of(tc * 128, 128)

        def issue(chunk, j):
            sem = sem_a if j % 2 == 0 else sem_b
            for k in range(G):
                s = chunk[j * G + k]
                slot = (j % 2) * G + k
                pltpu.async_copy(
                    t2_h.at[:, pl.ds(scol(s), 128)], stage.at[slot], sem)

        def wait_group(j):
            sem = sem_a if j % 2 == 0 else sem_b
            for _ in range(G):
                pltpu.make_async_copy(
                    t2_h.at[:, pl.ds(0, 128)], stage.at[0], sem).wait()

        def process(chunk, j, gbase):
            for k in range(G):
                s = chunk[j * G + k]
                slot = (j % 2) * G + k
                lane = jnp.minimum(s - scol(s), jnp.int32(127))
                lanev = jnp.full((L,), lane, jnp.int32)
                acc = None
                for c in range(D2 // L):
                    v = plsc.load_gather(
                        stage.at[slot], [c * L + iot, lanev])
                    term = v * wch[c]
                    acc = term if acc is None else acc + term
                accT[pl.ds((gbase + j * G + k) * L, L)] = acc

        chunk0 = idv[pl.ds(0, L)]
        issue(chunk0, 0)
        issue(chunk0, 1)

        def body(gg, carry):
            gbase = gg * L
            chunk = idv[pl.ds(gbase, L)]
            chunk_n = idv[pl.ds(gbase + L, L)]
            wait_group(0)
            process(chunk, 0, gbase)
            issue(chunk, 2)
            wait_group(1)
            process(chunk, 1, gbase)
            issue(chunk, 3)
            wait_group(0)
            process(chunk, 2, gbase)
            issue(chunk_n, 0)
            wait_group(1)
            process(chunk, 3, gbase)
            issue(chunk_n, 1)
            return carry

        lax.fori_loop(0, BPW // L, body, 0)
        wait_group(0)
        wait_group(1)

        pltpu.sync_copy(fd_h.at[pl.ds(base, BPW)], fdv)
        pltpu.sync_copy(b_h, bv)
        pltpu.make_async_copy(p0_h.at[idx0], g0, sem_g).wait()
        pltpu.make_async_copy(p1_h.at[idx1], g1, sem_g).wait()
        pltpu.make_async_copy(ptail_h.at[idxt], gt, sem_g).wait()
        bvec = bv[...]

        for g in range(BPW // L):
            gbase = row_base + g * (L * L)
            s = plsc.load_gather(accT, [gbase])
            for k in range(1, L):
                s = s + plsc.load_gather(accT, [gbase + k])
            sl = pl.ds(g * L, L)
            t2sel = jnp.where(idv[sl] >= TAILBASE, gt[sl], s)
            outv[sl] = g0[sl] + g1[sl] + t2sel + fdv[sl] + bvec
        pltpu.sync_copy(outv, out_h.at[pl.ds(base, BPW)])

    return sc_kernel


_SC_KERNEL = _build_sc()


def kernel(hierarchy_ids_level0, hierarchy_ids_level1, hierarchy_ids_level2,
           features, emb_level0, emb_level1, emb_level2, W, b):
    id0 = hierarchy_ids_level0.astype(jnp.int32)
    id1 = hierarchy_ids_level1.astype(jnp.int32)
    id2 = hierarchy_ids_level2.astype(jnp.int32)
    w0 = W[:, :D0]
    w1 = W[:, D0:D0 + D1]
    w2 = W[:, D0 + D1:D0 + D1 + D2]
    wf = W[:, D0 + D1 + D2:]
    t2t = emb_level2.T
    p0 = _MV0(w0, emb_level0.T)
    p1 = _MV1(w1, emb_level1.T)
    fd = _MVF(wf, features.T)
    ptail = _MVT(w2, t2t[:, TAILBASE:])
    b_vec = jnp.broadcast_to(b.astype(jnp.float32), (L,))
    return _SC_KERNEL(id0, id1, id2, t2t, w2.reshape(-1), p0, p1, ptail, fd,
                      b_vec)


# speedup vs baseline: 1.2013x; 1.2013x over previous
import functools

import jax
import jax.numpy as jnp
from jax import lax
from jax.experimental import pallas as pl
from jax.experimental.pallas import tpu as pltpu
from jax.experimental.pallas import tpu_sc as plsc

B = 4096
D0, D1, D2, DF = 32, 64, 64, 64
V0, V1, V2 = 1000, 100000, 1000000
NC, NS, L = 2, 16, 16
NW = NC * NS
BPW = B // NW
TAILBASE = (V2 // 128) * 128
NTAIL = V2 - TAILBASE
TCMAX = V2 // 128 - 1
G = 4
NG = BPW // G


def _matvec(d, v, blk):
    grid = (v + blk - 1) // blk

    def body(w_ref, t_ref, o_ref):
        o_ref[...] = jnp.dot(
            w_ref[...], t_ref[...], preferred_element_type=jnp.float32
        )[0]

    return pl.pallas_call(
        body,
        grid=(grid,),
        in_specs=[
            pl.BlockSpec((1, d), lambda i: (0, 0)),
            pl.BlockSpec((d, blk), lambda i: (0, i)),
        ],
        out_specs=pl.BlockSpec((blk,), lambda i: (i,)),
        out_shape=jax.ShapeDtypeStruct((v,), jnp.float32),
    )


_MV1 = _matvec(D1, V1, 32768)


def _small_dense_body(w0_ref, t0_ref, wt_ref, tt_ref, wf_ref, f_ref,
                      p0_ref, pt_ref, fd_ref):
    p0_ref[...] = jnp.dot(
        w0_ref[...], t0_ref[...], preferred_element_type=jnp.float32)[0]
    pt_ref[...] = jnp.dot(
        wt_ref[...], tt_ref[...], preferred_element_type=jnp.float32)[0]
    fd_ref[...] = jnp.dot(
        wf_ref[...], f_ref[...], preferred_element_type=jnp.float32)[0]


_SMALL_DENSE = pl.pallas_call(
    _small_dense_body,
    out_shape=(
        jax.ShapeDtypeStruct((V0,), jnp.float32),
        jax.ShapeDtypeStruct((NTAIL,), jnp.float32),
        jax.ShapeDtypeStruct((B,), jnp.float32),
    ),
)


def _build_t2dot():
    mesh = plsc.VectorSubcoreMesh(core_axis_name="c", subcore_axis_name="s")

    @functools.partial(
        pl.kernel,
        mesh=mesh,
        out_type=jax.ShapeDtypeStruct((B,), jnp.float32),
        compiler_params=pltpu.CompilerParams(
            needs_layout_passes=False, use_tc_tiling_on_sc=True),
        scratch_types=[
            pltpu.VMEM((BPW + L,), jnp.int32),
            pltpu.VMEM((2 * G, D2, 128), jnp.float32),
            pltpu.VMEM((D2,), jnp.float32),
            pltpu.VMEM((BPW * L,), jnp.float32),
            pltpu.VMEM((BPW,), jnp.float32),
            pltpu.SemaphoreType.DMA,
            pltpu.SemaphoreType.DMA,
        ],
    )
    def sc_kernel(id2_h, t2_h, w2_h, out_h, idv, stage, wv, accT, outv,
                  sem_a, sem_b):
        wid = lax.axis_index("s") * NC + lax.axis_index("c")
        base = wid * BPW
        pltpu.sync_copy(id2_h.at[pl.ds(base, BPW)], idv.at[pl.ds(0, BPW)])
        pltpu.sync_copy(w2_h, wv)
        wch = [wv[pl.ds(c * L, L)] for c in range(D2 // L)]
        iot = lax.iota(jnp.int32, L)
        row_base = iot * L

        def scol(s):
            tc = jnp.minimum(
                lax.shift_right_logical(s, 7), jnp.int32(TCMAX))
            return pl.multiple_of(tc * 128, 128)

        def issue(chunk, j):
            sem = sem_a if j % 2 == 0 else sem_b
            for k in range(G):
                s = chunk[j * G + k]
                slot = (j % 2) * G + k
                pltpu.async_copy(
                    t2_h.at[:, pl.ds(scol(s), 128)], stage.at[slot], sem)

        def wait_group(j):
            sem = sem_a if j % 2 == 0 else sem_b
            for _ in range(G):
                pltpu.make_async_copy(
                    t2_h.at[:, pl.ds(0, 128)], stage.at[0], sem).wait()

        def process(chunk, j, gbase):
            for k in range(G):
                s = chunk[j * G + k]
                slot = (j % 2) * G + k
                lane = jnp.minimum(s - scol(s), jnp.int32(127))
                lanev = jnp.full((L,), lane, jnp.int32)
                acc = None
                for c in range(D2 // L):
                    v = plsc.load_gather(
                        stage.at[slot], [c * L + iot, lanev])
                    term = v * wch[c]
                    acc = term if acc is None else acc + term
                accT[pl.ds((gbase + j * G + k) * L, L)] = acc

        chunk0 = idv[pl.ds(0, L)]
        issue(chunk0, 0)
        issue(chunk0, 1)

        def body(gg, carry):
            gbase = gg * L
            chunk = idv[pl.ds(gbase, L)]
            chunk_n = idv[pl.ds(gbase + L, L)]
            wait_group(0)
            process(chunk, 0, gbase)
            issue(chunk, 2)
            wait_group(1)
            process(chunk, 1, gbase)
            issue(chunk, 3)
            wait_group(0)
            process(chunk, 2, gbase)
            issue(chunk_n, 0)
            wait_group(1)
            process(chunk, 3, gbase)
            issue(chunk_n, 1)
            return carry

        lax.fori_loop(0, BPW // L, body, 0)
        wait_group(0)
        wait_group(1)

        for g in range(BPW // L):
            gbase = row_base + g * (L * L)
            s = plsc.load_gather(accT, [gbase])
            for k in range(1, L):
                s = s + plsc.load_gather(accT, [gbase + k])
            outv[pl.ds(g * L, L)] = s
        pltpu.sync_copy(outv, out_h.at[pl.ds(base, BPW)])

    return sc_kernel


def _build_assemble():
    mesh = plsc.VectorSubcoreMesh(core_axis_name="c", subcore_axis_name="s")

    @functools.partial(
        pl.kernel,
        mesh=mesh,
        out_type=jax.ShapeDtypeStruct((B,), jnp.float32),
        compiler_params=pltpu.CompilerParams(
            needs_layout_passes=False, use_tc_tiling_on_sc=False),
        scratch_types=[
            pltpu.VMEM((BPW,), jnp.int32),
            pltpu.VMEM((BPW,), jnp.int32),
            pltpu.VMEM((BPW,), jnp.int32),
            pltpu.VMEM((BPW,), jnp.int32),
            pltpu.VMEM((BPW,), jnp.float32),
            pltpu.VMEM((BPW,), jnp.float32),
            pltpu.VMEM((BPW,), jnp.float32),
            pltpu.VMEM((BPW,), jnp.float32),
            pltpu.VMEM((BPW,), jnp.float32),
            pltpu.VMEM((L,), jnp.float32),
            pltpu.VMEM((BPW,), jnp.float32),
            pltpu.SemaphoreType.DMA,
        ],
    )
    def sc_kernel(id0_h, id1_h, id2_h, p0_h, p1_h, ptail_h, t2_h, fd_h, b_h,
                  out_h, idx0, idx1, idx2, idxt, g0, g1, gt, t2v, fdv, bv,
                  outv, sem):
        wid = lax.axis_index("s") * NC + lax.axis_index("c")
        base = wid * BPW
        pltpu.sync_copy(id0_h.at[pl.ds(base, BPW)], idx0)
        pltpu.sync_copy(id1_h.at[pl.ds(base, BPW)], idx1)
        pltpu.sync_copy(id2_h.at[pl.ds(base, BPW)], idx2)
        for c in range(BPW // L):
            sl = pl.ds(c * L, L)
            idxt[sl] = jnp.clip(idx2[sl] - TAILBASE, 0, NTAIL - 1)
        cp0 = pltpu.async_copy(p0_h.at[idx0], g0, sem)
        cp1 = pltpu.async_copy(p1_h.at[idx1], g1, sem)
        cpt = pltpu.async_copy(ptail_h.at[idxt], gt, sem)
        pltpu.sync_copy(t2_h.at[pl.ds(base, BPW)], t2v)
        pltpu.sync_copy(fd_h.at[pl.ds(base, BPW)], fdv)
        pltpu.sync_copy(b_h, bv)
        cp0.wait()
        cp1.wait()
        cpt.wait()
        bvec = bv[...]
        for c in range(BPW // L):
            sl = pl.ds(c * L, L)
            t2sel = jnp.where(idx2[sl] >= TAILBASE, gt[sl], t2v[sl])
            outv[sl] = g0[sl] + g1[sl] + t2sel + fdv[sl] + bvec
        pltpu.sync_copy(outv, out_h.at[pl.ds(base, BPW)])

    return sc_kernel


_SC_T2DOT = _build_t2dot()
_SC_ASSEMBLE = _build_assemble()


def kernel(hierarchy_ids_level0, hierarchy_ids_level1, hierarchy_ids_level2,
           features, emb_level0, emb_level1, emb_level2, W, b):
    id0 = hierarchy_ids_level0.astype(jnp.int32)
    id1 = hierarchy_ids_level1.astype(jnp.int32)
    id2 = hierarchy_ids_level2.astype(jnp.int32)
    w0 = W[:, :D0]
    w1 = W[:, D0:D0 + D1]
    w2 = W[:, D0 + D1:D0 + D1 + D2]
    wf = W[:, D0 + D1 + D2:]
    t2t = emb_level2.T
    p1 = _MV1(w1, emb_level1.T)
    p0, ptail, fd = _SMALL_DENSE(w0, emb_level0.T, w2, t2t[:, TAILBASE:],
                                 wf, features.T)
    t2dot = _SC_T2DOT(id2, t2t, w2.reshape(-1))
    b_vec = jnp.broadcast_to(b.astype(jnp.float32), (L,))
    return _SC_ASSEMBLE(id0, id1, id2, p0, p1, ptail, t2dot, fd, b_vec)
